# Initial kernel scaffold; baseline (speedup 1.0000x reference)
import functools

import jax
import jax.numpy as jnp
from jax import lax
from jax.experimental import pallas as pl
from jax.experimental.pallas import tpu as pltpu
from jax.experimental.pallas import tpu_sc as plsc

N = 10000
E = 320000
D = 128

NC = 2
NS = 16
NW = NC * NS
LANES = 16

NPAD = 10240
K = 80
EPW = E // NW
CHUNKS = EPW // K
ROWS_PS = NPAD // NS
RBLK = 2048
GRID = NPAD // RBLK

_mesh = plsc.VectorSubcoreMesh(core_axis_name="c", subcore_axis_name="s")

_F32 = jnp.float32
_PREC = jax.lax.Precision.HIGHEST


@functools.partial(
    pl.kernel,
    out_type=jax.ShapeDtypeStruct((NC, NPAD), _F32),
    mesh=_mesh,
    scratch_types=[
        pltpu.VMEM_SHARED((NPAD,), _F32),
        pltpu.VMEM((CHUNKS, K), jnp.int32),
        pltpu.VMEM((K,), _F32),
        pltpu.VMEM((ROWS_PS,), _F32),
    ],
)
def _sc_degree(dst_hbm, out_hbm, deg_sh, didx, ones_v, zeros_v):
    c = lax.axis_index("c")
    s = lax.axis_index("s")
    wid = s * NC + c

    @pl.loop(0, ROWS_PS // LANES)
    def _(i):
        zeros_v[pl.ds(i * LANES, LANES)] = jnp.zeros((LANES,), _F32)

    @pl.loop(0, K // LANES)
    def _(i):
        ones_v[pl.ds(i * LANES, LANES)] = jnp.ones((LANES,), _F32)

    pltpu.sync_copy(zeros_v, deg_sh.at[pl.ds(s * ROWS_PS, ROWS_PS)])
    plsc.subcore_barrier()

    pltpu.sync_copy(dst_hbm.at[pl.ds(wid * CHUNKS, CHUNKS)], didx)

    @pl.loop(0, CHUNKS)
    def _(j):
        pltpu.sync_copy(ones_v, deg_sh.at[didx.at[j]], add=True)

    plsc.subcore_barrier()
    pltpu.sync_copy(deg_sh.at[pl.ds(s * ROWS_PS, ROWS_PS)],
                    out_hbm.at[c].at[pl.ds(s * ROWS_PS, ROWS_PS)])


@functools.partial(
    pl.kernel,
    out_type=jax.ShapeDtypeStruct((NC, NPAD, D), _F32),
    mesh=_mesh,
    scratch_types=[
        pltpu.VMEM_SHARED((NPAD, D), _F32),
        pltpu.VMEM((CHUNKS, K), jnp.int32),
        pltpu.VMEM((CHUNKS, K), jnp.int32),
        pltpu.VMEM((K, D), _F32),
        pltpu.SemaphoreType.DMA,
    ],
)
def _sc_aggregate(hs_hbm, src_hbm, dst_hbm, out_hbm, agg_sh, sidx, didx,
                  rows, sem):
    c = lax.axis_index("c")
    s = lax.axis_index("s")
    wid = s * NC + c

    @pl.loop(0, (K * D) // LANES)
    def _(t):
        r = t // (D // LANES)
        q = t % (D // LANES)
        rows[r, pl.ds(q * LANES, LANES)] = jnp.zeros((LANES,), _F32)

    @pl.loop(0, ROWS_PS // K)
    def _(i):
        pltpu.sync_copy(rows, agg_sh.at[pl.ds(s * ROWS_PS + i * K, K)])

    plsc.subcore_barrier()

    pltpu.sync_copy(src_hbm.at[pl.ds(wid * CHUNKS, CHUNKS)], sidx)
    pltpu.sync_copy(dst_hbm.at[pl.ds(wid * CHUNKS, CHUNKS)], didx)

    @pl.loop(0, CHUNKS)
    def _(j):
        pltpu.async_copy(hs_hbm.at[sidx.at[j]], rows, sem).wait()
        pltpu.sync_copy(rows, agg_sh.at[didx.at[j]], add=True)

    plsc.subcore_barrier()

    @pl.loop(0, ROWS_PS // K)
    def _(i):
        pltpu.sync_copy(agg_sh.at[pl.ds(s * ROWS_PS + i * K, K)],
                        out_hbm.at[c].at[pl.ds(s * ROWS_PS + i * K, K)])


def _pre_body(x_ref, w_ref, b_ref, dis_ref, h_ref, hs_ref):
    h = jnp.dot(x_ref[...], w_ref[...], precision=_PREC,
                preferred_element_type=_F32)
    h = jnp.maximum(h + b_ref[...], 0.0)
    h_ref[...] = h
    hs_ref[...] = h * dis_ref[...]


def _mid_body(aggp_ref, dis_ref, h_ref, w_ref, b_ref, hn_ref, hs_ref):
    agg = (aggp_ref[0] + aggp_ref[1]) * dis_ref[...]
    hn = jnp.dot(agg, w_ref[...], precision=_PREC, preferred_element_type=_F32)
    hn = jnp.maximum(hn + b_ref[...] + h_ref[...], 0.0)
    hn_ref[...] = hn
    hs_ref[...] = hn * dis_ref[...]


def _final_body(aggp_ref, dis_ref, h_ref, w_ref, b_ref, wh_ref, bh_ref,
                out_ref):
    agg = (aggp_ref[0] + aggp_ref[1]) * dis_ref[...]
    h3 = jnp.dot(agg, w_ref[...], precision=_PREC, preferred_element_type=_F32)
    h3 = jnp.maximum(h3 + b_ref[...] + h_ref[...], 0.0)
    out_ref[...] = jnp.dot(h3, wh_ref[...], precision=_PREC,
                           preferred_element_type=_F32) + bh_ref[...]


_row_spec = pl.BlockSpec((RBLK, D), lambda i: (i, 0))
_mat_spec = pl.BlockSpec((D, D), lambda i: (0, 0))
_bias_spec = pl.BlockSpec((1, D), lambda i: (0, 0))
_aggp_spec = pl.BlockSpec((NC, RBLK, D), lambda i: (0, i, 0))
_nd_shape = jax.ShapeDtypeStruct((NPAD, D), _F32)


_tc_pre = pl.pallas_call(
    _pre_body,
    grid=(GRID,),
    in_specs=[_row_spec, _mat_spec, _bias_spec, _row_spec],
    out_specs=[_row_spec, _row_spec],
    out_shape=[_nd_shape, _nd_shape],
)

_tc_mid = pl.pallas_call(
    _mid_body,
    grid=(GRID,),
    in_specs=[_aggp_spec, _row_spec, _row_spec, _mat_spec, _bias_spec],
    out_specs=[_row_spec, _row_spec],
    out_shape=[_nd_shape, _nd_shape],
)

_tc_final = pl.pallas_call(
    _final_body,
    grid=(GRID,),
    in_specs=[_aggp_spec, _row_spec, _row_spec, _mat_spec, _bias_spec,
              _mat_spec, _bias_spec],
    out_specs=_row_spec,
    out_shape=_nd_shape,
)


def kernel(x, edge_index, W_pre, b_pre, W1, b1, W2, b2, W3, b3, W_head,
           b_head):
    src2 = edge_index[0].reshape(E // K, K)
    dst2 = edge_index[1].reshape(E // K, K)

    degp = _sc_degree(dst2)
    dis = jax.lax.rsqrt(jnp.maximum(degp[0] + degp[1], 1.0))
    dis_full = jnp.broadcast_to(dis[:, None], (NPAD, D))

    x_p = jnp.pad(x, ((0, NPAD - N), (0, 0)))
    h, hs = _tc_pre(x_p, W_pre, b_pre.reshape(1, D), dis_full)

    for W, b in ((W1, b1), (W2, b2)):
        aggp = _sc_aggregate(hs, src2, dst2)
        h, hs = _tc_mid(aggp, dis_full, h, W, b.reshape(1, D))

    aggp = _sc_aggregate(hs, src2, dst2)
    out = _tc_final(aggp, dis_full, h, W3, b3.reshape(1, D), W_head,
                    b_head.reshape(1, D))
    return out[:N]


# speedup vs baseline: 13.8869x; 13.8869x over previous
import functools

import jax
import jax.numpy as jnp
from jax import lax
from jax.experimental import pallas as pl
from jax.experimental.pallas import tpu as pltpu
from jax.experimental.pallas import tpu_sc as plsc

N = 10000
E = 320000
D = 128

NC = 2
NS = 16
NW = NC * NS
LANES = 16

NPAD = 10240
K = 80
EPW = E // NW
CHUNKS = EPW // K
ROWS_PS = NPAD // NS
RBLK = 2048
GRID = NPAD // RBLK

_mesh = plsc.VectorSubcoreMesh(core_axis_name="c", subcore_axis_name="s")

_F32 = jnp.float32
_PREC = jax.lax.Precision.HIGHEST


@functools.partial(
    pl.kernel,
    out_type=[jax.ShapeDtypeStruct((NPAD,), _F32),
              jax.ShapeDtypeStruct((NPAD,), _F32)],
    mesh=_mesh,
    scratch_types=[
        pltpu.VMEM_SHARED((NPAD,), _F32),
        pltpu.VMEM((CHUNKS, K), jnp.int32),
        pltpu.VMEM((K,), _F32),
        pltpu.VMEM((ROWS_PS,), _F32),
    ],
)
def _sc_degree(dst_hbm, out0_hbm, out1_hbm, deg_sh, didx, ones_v, zeros_v):
    c = lax.axis_index("c")
    s = lax.axis_index("s")
    wid = s * NC + c

    @pl.loop(0, ROWS_PS // LANES)
    def _(i):
        zeros_v[pl.ds(i * LANES, LANES)] = jnp.zeros((LANES,), _F32)

    @pl.loop(0, K // LANES)
    def _(i):
        ones_v[pl.ds(i * LANES, LANES)] = jnp.ones((LANES,), _F32)

    pltpu.sync_copy(zeros_v, deg_sh.at[pl.ds(s * ROWS_PS, ROWS_PS)])
    plsc.subcore_barrier()

    pltpu.sync_copy(dst_hbm.at[wid], didx)

    @pl.loop(0, CHUNKS)
    def _(j):
        pltpu.sync_copy(ones_v, deg_sh.at[didx.at[j]], add=True)

    plsc.subcore_barrier()

    @pl.when(c == 0)
    def _():
        pltpu.sync_copy(deg_sh.at[pl.ds(s * ROWS_PS, ROWS_PS)],
                        out0_hbm.at[pl.ds(s * ROWS_PS, ROWS_PS)])

    @pl.when(c == 1)
    def _():
        pltpu.sync_copy(deg_sh.at[pl.ds(s * ROWS_PS, ROWS_PS)],
                        out1_hbm.at[pl.ds(s * ROWS_PS, ROWS_PS)])


@functools.partial(
    pl.kernel,
    out_type=jax.ShapeDtypeStruct((NC, NPAD, D), _F32),
    mesh=_mesh,
    scratch_types=[
        pltpu.VMEM_SHARED((NPAD, D), _F32),
        pltpu.VMEM((CHUNKS, K), jnp.int32),
        pltpu.VMEM((CHUNKS, K), jnp.int32),
        pltpu.VMEM((K, D), _F32),
        pltpu.SemaphoreType.DMA,
    ],
)
def _sc_aggregate(hs_hbm, src_hbm, dst_hbm, out_hbm, agg_sh, sidx, didx,
                  rows, sem):
    c = lax.axis_index("c")
    s = lax.axis_index("s")
    wid = s * NC + c

    @pl.loop(0, (K * D) // LANES)
    def _(t):
        r = t // (D // LANES)
        q = t % (D // LANES)
        rows[r, pl.ds(q * LANES, LANES)] = jnp.zeros((LANES,), _F32)

    @pl.loop(0, ROWS_PS // K)
    def _(i):
        pltpu.sync_copy(rows, agg_sh.at[pl.ds(s * ROWS_PS + i * K, K)])

    plsc.subcore_barrier()

    pltpu.sync_copy(src_hbm.at[wid], sidx)
    pltpu.sync_copy(dst_hbm.at[wid], didx)

    @pl.loop(0, CHUNKS)
    def _(j):
        pltpu.async_copy(hs_hbm.at[sidx.at[j]], rows, sem).wait()
        pltpu.sync_copy(rows, agg_sh.at[didx.at[j]], add=True)

    plsc.subcore_barrier()

    @pl.loop(0, ROWS_PS // K)
    def _(i):
        pltpu.sync_copy(agg_sh.at[pl.ds(s * ROWS_PS + i * K, K)],
                        out_hbm.at[c].at[pl.ds(s * ROWS_PS + i * K, K)])


def _pre_body(x_ref, w_ref, b_ref, dis_ref, h_ref, hs_ref):
    h = jnp.dot(x_ref[...], w_ref[...], precision=_PREC,
                preferred_element_type=_F32)
    h = jnp.maximum(h + b_ref[...], 0.0)
    h_ref[...] = h
    hs_ref[...] = h * dis_ref[...]


def _mid_body(aggp_ref, dis_ref, h_ref, w_ref, b_ref, hn_ref, hs_ref):
    agg = (aggp_ref[0] + aggp_ref[1]) * dis_ref[...]
    hn = jnp.dot(agg, w_ref[...], precision=_PREC, preferred_element_type=_F32)
    hn = jnp.maximum(hn + b_ref[...] + h_ref[...], 0.0)
    hn_ref[...] = hn
    hs_ref[...] = hn * dis_ref[...]


def _final_body(aggp_ref, dis_ref, h_ref, w_ref, b_ref, wh_ref, bh_ref,
                out_ref):
    agg = (aggp_ref[0] + aggp_ref[1]) * dis_ref[...]
    h3 = jnp.dot(agg, w_ref[...], precision=_PREC, preferred_element_type=_F32)
    h3 = jnp.maximum(h3 + b_ref[...] + h_ref[...], 0.0)
    out_ref[...] = jnp.dot(h3, wh_ref[...], precision=_PREC,
                           preferred_element_type=_F32) + bh_ref[...]


_row_spec = pl.BlockSpec((RBLK, D), lambda i: (i, 0))
_mat_spec = pl.BlockSpec((D, D), lambda i: (0, 0))
_bias_spec = pl.BlockSpec((1, D), lambda i: (0, 0))
_aggp_spec = pl.BlockSpec((NC, RBLK, D), lambda i: (0, i, 0))
_nd_shape = jax.ShapeDtypeStruct((NPAD, D), _F32)


_tc_pre = pl.pallas_call(
    _pre_body,
    grid=(GRID,),
    in_specs=[_row_spec, _mat_spec, _bias_spec, _row_spec],
    out_specs=[_row_spec, _row_spec],
    out_shape=[_nd_shape, _nd_shape],
)

_tc_mid = pl.pallas_call(
    _mid_body,
    grid=(GRID,),
    in_specs=[_aggp_spec, _row_spec, _row_spec, _mat_spec, _bias_spec],
    out_specs=[_row_spec, _row_spec],
    out_shape=[_nd_shape, _nd_shape],
)

_tc_final = pl.pallas_call(
    _final_body,
    grid=(GRID,),
    in_specs=[_aggp_spec, _row_spec, _row_spec, _mat_spec, _bias_spec,
              _mat_spec, _bias_spec],
    out_specs=_row_spec,
    out_shape=_nd_shape,
)


def kernel(x, edge_index, W_pre, b_pre, W1, b1, W2, b2, W3, b3, W_head,
           b_head):
    src2 = edge_index[0].reshape(NW, CHUNKS, K)
    dst2 = edge_index[1].reshape(NW, CHUNKS, K)

    deg0, deg1 = _sc_degree(dst2)
    dis = jax.lax.rsqrt(jnp.maximum(deg0 + deg1, 1.0))
    dis_full = jnp.broadcast_to(dis[:, None], (NPAD, D))

    x_p = jnp.pad(x, ((0, NPAD - N), (0, 0)))
    h, hs = _tc_pre(x_p, W_pre, b_pre.reshape(1, D), dis_full)

    for W, b in ((W1, b1), (W2, b2)):
        aggp = _sc_aggregate(hs, src2, dst2)
        h, hs = _tc_mid(aggp, dis_full, h, W, b.reshape(1, D))

    aggp = _sc_aggregate(hs, src2, dst2)
    out = _tc_final(aggp, dis_full, h, W3, b3.reshape(1, D), W_head,
                    b_head.reshape(1, D))
    return out[:N]
